# baseline (speedup 1.0000x reference)
import jax
import jax.numpy as jnp
from jax.experimental import pallas as pl
from jax.experimental.pallas import tpu as pltpu

_N = 100000
_I = 10
_H = 128
_BN = 5000
_NEG = -3.0e38
_POS = 3.0e38
_INTERPRET = False


def _stage1_body(pc_ref, mk_ref, w1_ref, b1_ref, w2_ref, b2_ref, w3_ref, b3_ref,
                 m_ref, t_ref, yaw_ref, aux_ref, vmax_ref, vmin_ref,
                 smax_ref, smin_ref):
    i = pl.program_id(0)
    nb = pl.num_programs(0)
    x = pc_ref[...]
    mk = mk_ref[...]

    h = jnp.maximum(
        jnp.dot(x, w1_ref[...], preferred_element_type=jnp.float32)
        + b1_ref[...][None, :], 0.0)
    h = jnp.maximum(
        jnp.dot(h, w2_ref[...], preferred_element_type=jnp.float32)
        + b2_ref[...][None, :], 0.0)
    out4 = (jnp.dot(h, w3_ref[...], preferred_element_type=jnp.float32)
            + b3_ref[...][None, :])
    t_ref[...] = out4[:, :3]
    yaw_ref[...] = out4[:, 3:4]

    mmax = jnp.max(mk, axis=1, keepdims=True)
    sh = mk - mmax
    e = jnp.exp(sh)
    s = jnp.sum(e, axis=1, keepdims=True)
    m_ref[...] = jnp.exp(sh - jnp.log(s))

    bn = x.shape[0]
    mkT = jnp.transpose(mk)
    pcT = jnp.transpose(x)
    out4T = jnp.transpose(out4)
    mmaxT = jnp.max(mkT, axis=0, keepdims=True)
    subi = jax.lax.broadcasted_iota(jnp.int32, (_I, bn), 0)
    big = jnp.int32(2 ** 30)
    indT = jnp.min(jnp.where(mkT == mmaxT, subi, big),
                   axis=0, keepdims=True)
    indTf = jax.lax.bitcast_convert_type(indT, jnp.float32)
    aux_ref[0] = jnp.concatenate([out4T, pcT, indTf], axis=0)

    pc8x = jnp.concatenate([pcT, jnp.full((5, bn), _NEG, jnp.float32)], axis=0)
    pc8n = jnp.concatenate([pcT, jnp.full((5, bn), _POS, jnp.float32)], axis=0)

    @pl.when(i == 0)
    def _init():
        smax_ref[...] = jnp.full(smax_ref.shape, _NEG, jnp.float32)
        smin_ref[...] = jnp.full(smin_ref.shape, _POS, jnp.float32)

    for inst in range(_I):
        cond8 = jnp.broadcast_to(indT == inst, (8, bn))
        smax_ref[inst] = jnp.maximum(smax_ref[inst],
                                     jnp.where(cond8, pc8x, _NEG))
        smin_ref[inst] = jnp.minimum(smin_ref[inst],
                                     jnp.where(cond8, pc8n, _POS))

    @pl.when(i == nb - 1)
    def _emit():
        mxs = [jnp.max(smax_ref[inst], axis=1, keepdims=True)
               for inst in range(_I)]
        mns = [jnp.min(smin_ref[inst], axis=1, keepdims=True)
               for inst in range(_I)]
        vmax_ref[0:8, 0:_I] = jnp.concatenate(mxs, axis=1)
        vmin_ref[0:8, 0:_I] = jnp.concatenate(mns, axis=1)


def _stage2_body(aux_ref, vmax_ref, vmin_ref, flow_ref):
    ax = aux_ref[0]
    z = ax[3:4, :]
    px = ax[4:5, :]
    py = ax[5:6, :]
    pz = ax[6:7, :]
    ind = jax.lax.bitcast_convert_type(ax[7:8, :], jnp.int32)

    zero = jnp.zeros_like(z)
    accx, accy, accz = zero, zero, zero

    def _center(c, i):
        mx = vmax_ref[c, i]
        mn = vmin_ref[c, i]
        return jnp.where(mx >= mn, (mx + mn) * 0.5, 0.0)

    for i in range(_I):
        cond = ind == i
        accx = jnp.where(cond, _center(0, i), accx)
        accy = jnp.where(cond, _center(1, i), accy)
        accz = jnp.where(cond, _center(2, i), accz)

    dx = px - accx
    dy = py - accy
    dz = pz - accz

    a2 = z * z
    a = jnp.sqrt(a2 + 1e-12)
    small = a < 1e-4
    a_safe = jnp.where(small, 1.0, a)
    sin_term = jnp.where(small, 1.0 - a2 / 6.0, jnp.sin(a_safe) / a_safe)
    cos_term = jnp.where(small, 0.5 - a2 / 24.0,
                         (1.0 - jnp.cos(a_safe)) / (a_safe * a_safe))
    s = sin_term * z
    cm = 1.0 - cos_term * a2

    rx = cm * dx - s * dy
    ry = s * dx + cm * dy
    fx = (rx + accx + ax[0:1, :]) - px
    fy = (ry + accy + ax[1:2, :]) - py
    fz = (dz + accz + ax[2:3, :]) - pz
    flowT = jnp.concatenate([fx, fy, fz], axis=0)
    flow_ref[...] = jnp.transpose(flowT)


def kernel(pc1, mask, W1, b1, W2, b2, W3, b3):
    pc = pc1.reshape(_N, 3)
    mk = mask.reshape(_N, _I)
    nb = _N // _BN
    grid = (nb,)

    m, t, yaw, aux, vmax, vmin = pl.pallas_call(
        _stage1_body,
        grid=grid,
        in_specs=[
            pl.BlockSpec((_BN, 3), lambda i: (i, 0)),
            pl.BlockSpec((_BN, _I), lambda i: (i, 0)),
            pl.BlockSpec((3, _H), lambda i: (0, 0)),
            pl.BlockSpec((_H,), lambda i: (0,)),
            pl.BlockSpec((_H, _H), lambda i: (0, 0)),
            pl.BlockSpec((_H,), lambda i: (0,)),
            pl.BlockSpec((_H, 4), lambda i: (0, 0)),
            pl.BlockSpec((4,), lambda i: (0,)),
        ],
        out_specs=[
            pl.BlockSpec((_BN, _I), lambda i: (i, 0)),
            pl.BlockSpec((_BN, 3), lambda i: (i, 0)),
            pl.BlockSpec((_BN, 1), lambda i: (i, 0)),
            pl.BlockSpec((1, 8, _BN), lambda i: (i, 0, 0)),
            pl.BlockSpec((8, 128), lambda i: (0, 0)),
            pl.BlockSpec((8, 128), lambda i: (0, 0)),
        ],
        out_shape=[
            jax.ShapeDtypeStruct((_N, _I), jnp.float32),
            jax.ShapeDtypeStruct((_N, 3), jnp.float32),
            jax.ShapeDtypeStruct((_N, 1), jnp.float32),
            jax.ShapeDtypeStruct((nb, 8, _BN), jnp.float32),
            jax.ShapeDtypeStruct((8, 128), jnp.float32),
            jax.ShapeDtypeStruct((8, 128), jnp.float32),
        ],
        scratch_shapes=[
            pltpu.VMEM((_I, 8, _BN), jnp.float32),
            pltpu.VMEM((_I, 8, _BN), jnp.float32),
        ],
        interpret=_INTERPRET,
    )(pc, mk, W1, b1, W2, b2, W3, b3)

    flow = pl.pallas_call(
        _stage2_body,
        grid=grid,
        in_specs=[
            pl.BlockSpec((1, 8, _BN), lambda i: (i, 0, 0)),
            pl.BlockSpec((8, 128), lambda i: (0, 0)),
            pl.BlockSpec((8, 128), lambda i: (0, 0)),
        ],
        out_specs=pl.BlockSpec((_BN, 3), lambda i: (i, 0)),
        out_shape=jax.ShapeDtypeStruct((_N, 3), jnp.float32),
        interpret=_INTERPRET,
    )(aux, vmax, vmin)

    return (flow.reshape(1, _N, 3), m.reshape(1, _N, _I),
            t.reshape(1, _N, 3), yaw.reshape(1, _N, 1))


# speedup vs baseline: 1.3890x; 1.3890x over previous
import functools

import jax
import jax.numpy as jnp
from jax import lax
from jax.experimental import pallas as pl
from jax.experimental.pallas import tpu as pltpu
from jax.experimental.pallas import tpu_sc as plsc

_N = 100000
_I = 10
_H = 128
_BN = 4000
_NB = _N // _BN
_NW = 32
_NEG = -3.0e38
_POS = 3.0e38
_INTERPRET = False


def _stage1_body(pc_ref, mk_ref, w1_ref, b1_ref, w2_ref, b2_ref, w3_ref, b3_ref,
                 m_ref, t_ref, yaw_ref, aux_ref):
    x = pc_ref[...]
    mk = mk_ref[...]

    h = jnp.maximum(
        jnp.dot(x, w1_ref[...], preferred_element_type=jnp.float32)
        + b1_ref[...][None, :], 0.0)
    h = jnp.maximum(
        jnp.dot(h, w2_ref[...], preferred_element_type=jnp.float32)
        + b2_ref[...][None, :], 0.0)
    out4 = (jnp.dot(h, w3_ref[...], preferred_element_type=jnp.float32)
            + b3_ref[...][None, :])
    t_ref[...] = out4[:, :3]
    yaw_ref[...] = out4[:, 3:4]

    mmax = jnp.max(mk, axis=1, keepdims=True)
    sh = mk - mmax
    e = jnp.exp(sh)
    s = jnp.sum(e, axis=1, keepdims=True)
    m_ref[...] = jnp.exp(sh - jnp.log(s))

    bn = x.shape[0]
    mkT = jnp.transpose(mk)
    pcT = jnp.transpose(x)
    out4T = jnp.transpose(out4)
    mmaxT = jnp.max(mkT, axis=0, keepdims=True)
    subi = jax.lax.broadcasted_iota(jnp.int32, (_I, bn), 0)
    big = jnp.int32(2 ** 30)
    indT = jnp.min(jnp.where(mkT == mmaxT, subi, big),
                   axis=0, keepdims=True)
    indTf = indT.astype(jnp.float32)
    aux_ref[0] = jnp.concatenate([out4T, pcT, indTf], axis=0)


def _seg_sc_body(aux_hbm, pmax_hbm, pmin_hbm, buf, omax, omin):
    cid = lax.axis_index("c")
    sid = lax.axis_index("s")
    w = sid * 2 + cid

    for inst in range(_I):
        for cc in range(3):
            omax[inst, cc, :] = jnp.full((16,), _NEG, jnp.float32)
            omin[inst, cc, :] = jnp.full((16,), _POS, jnp.float32)

    @pl.when(w < _NB)
    def _work():
        pltpu.sync_copy(aux_hbm.at[w], buf)

        for pair in range(_I // 2):
            i0 = 2 * pair
            i1 = i0 + 1

            def body(j, carry):
                (ax0, ay0, az0, nx0, ny0, nz0,
                 ax1, ay1, az1, nx1, ny1, nz1) = carry
                base = j * 16
                vx = buf[4, pl.ds(base, 16)]
                vy = buf[5, pl.ds(base, 16)]
                vz = buf[6, pl.ds(base, 16)]
                vi = buf[7, pl.ds(base, 16)]
                c0 = vi == float(i0)
                c1 = vi == float(i1)
                ax0 = jnp.maximum(ax0, jnp.where(c0, vx, _NEG))
                ay0 = jnp.maximum(ay0, jnp.where(c0, vy, _NEG))
                az0 = jnp.maximum(az0, jnp.where(c0, vz, _NEG))
                nx0 = jnp.minimum(nx0, jnp.where(c0, vx, _POS))
                ny0 = jnp.minimum(ny0, jnp.where(c0, vy, _POS))
                nz0 = jnp.minimum(nz0, jnp.where(c0, vz, _POS))
                ax1 = jnp.maximum(ax1, jnp.where(c1, vx, _NEG))
                ay1 = jnp.maximum(ay1, jnp.where(c1, vy, _NEG))
                az1 = jnp.maximum(az1, jnp.where(c1, vz, _NEG))
                nx1 = jnp.minimum(nx1, jnp.where(c1, vx, _POS))
                ny1 = jnp.minimum(ny1, jnp.where(c1, vy, _POS))
                nz1 = jnp.minimum(nz1, jnp.where(c1, vz, _POS))
                return (ax0, ay0, az0, nx0, ny0, nz0,
                        ax1, ay1, az1, nx1, ny1, nz1)

            neg = jnp.full((16,), _NEG, jnp.float32)
            pos = jnp.full((16,), _POS, jnp.float32)
            res = lax.fori_loop(
                0, _BN // 16, body,
                (neg, neg, neg, pos, pos, pos,
                 neg, neg, neg, pos, pos, pos))
            omax[i0, 0, :] = res[0]
            omax[i0, 1, :] = res[1]
            omax[i0, 2, :] = res[2]
            omin[i0, 0, :] = res[3]
            omin[i0, 1, :] = res[4]
            omin[i0, 2, :] = res[5]
            omax[i1, 0, :] = res[6]
            omax[i1, 1, :] = res[7]
            omax[i1, 2, :] = res[8]
            omin[i1, 0, :] = res[9]
            omin[i1, 1, :] = res[10]
            omin[i1, 2, :] = res[11]

    pltpu.sync_copy(omax, pmax_hbm.at[w])
    pltpu.sync_copy(omin, pmin_hbm.at[w])


def _segment_sc(aux):
    mesh = plsc.VectorSubcoreMesh(core_axis_name="c", subcore_axis_name="s")
    fn = functools.partial(
        pl.kernel, mesh=mesh,
        out_type=[
            jax.ShapeDtypeStruct((_NW, _I, 3, 16), jnp.float32),
            jax.ShapeDtypeStruct((_NW, _I, 3, 16), jnp.float32),
        ],
        scratch_types=[
            pltpu.VMEM((8, _BN), jnp.float32),
            pltpu.VMEM((_I, 3, 16), jnp.float32),
            pltpu.VMEM((_I, 3, 16), jnp.float32),
        ],
    )(_seg_sc_body)
    return fn(aux)


def _stage2_body(aux_ref, pmax_ref, pmin_ref, flow_ref):
    ax = aux_ref[0]
    z = ax[3:4, :]
    px = ax[4:5, :]
    py = ax[5:6, :]
    pz = ax[6:7, :]
    ind = ax[7:8, :]

    rmax = pmax_ref[0]
    rmin = pmin_ref[0]
    for wx in range(1, _NW):
        rmax = jnp.maximum(rmax, pmax_ref[wx])
        rmin = jnp.minimum(rmin, pmin_ref[wx])
    vmax = jnp.max(rmax, axis=2)
    vmin = jnp.min(rmin, axis=2)

    zero = jnp.zeros_like(z)
    accx, accy, accz = zero, zero, zero

    def _center(c, i):
        mx = vmax[i, c]
        mn = vmin[i, c]
        return jnp.where(mx >= mn, (mx + mn) * 0.5, 0.0)

    for i in range(_I):
        cond = ind == float(i)
        accx = jnp.where(cond, _center(0, i), accx)
        accy = jnp.where(cond, _center(1, i), accy)
        accz = jnp.where(cond, _center(2, i), accz)

    dx = px - accx
    dy = py - accy
    dz = pz - accz

    a2 = z * z
    a = jnp.sqrt(a2 + 1e-12)
    small = a < 1e-4
    a_safe = jnp.where(small, 1.0, a)
    sin_term = jnp.where(small, 1.0 - a2 / 6.0, jnp.sin(a_safe) / a_safe)
    cos_term = jnp.where(small, 0.5 - a2 / 24.0,
                         (1.0 - jnp.cos(a_safe)) / (a_safe * a_safe))
    s = sin_term * z
    cm = 1.0 - cos_term * a2

    rx = cm * dx - s * dy
    ry = s * dx + cm * dy
    fx = (rx + accx + ax[0:1, :]) - px
    fy = (ry + accy + ax[1:2, :]) - py
    fz = (dz + accz + ax[2:3, :]) - pz
    flowT = jnp.concatenate([fx, fy, fz], axis=0)
    flow_ref[...] = jnp.transpose(flowT)


def kernel(pc1, mask, W1, b1, W2, b2, W3, b3):
    pc = pc1.reshape(_N, 3)
    mk = mask.reshape(_N, _I)
    grid = (_NB,)

    m, t, yaw, aux = pl.pallas_call(
        _stage1_body,
        grid=grid,
        in_specs=[
            pl.BlockSpec((_BN, 3), lambda i: (i, 0)),
            pl.BlockSpec((_BN, _I), lambda i: (i, 0)),
            pl.BlockSpec((3, _H), lambda i: (0, 0)),
            pl.BlockSpec((_H,), lambda i: (0,)),
            pl.BlockSpec((_H, _H), lambda i: (0, 0)),
            pl.BlockSpec((_H,), lambda i: (0,)),
            pl.BlockSpec((_H, 4), lambda i: (0, 0)),
            pl.BlockSpec((4,), lambda i: (0,)),
        ],
        out_specs=[
            pl.BlockSpec((_BN, _I), lambda i: (i, 0)),
            pl.BlockSpec((_BN, 3), lambda i: (i, 0)),
            pl.BlockSpec((_BN, 1), lambda i: (i, 0)),
            pl.BlockSpec((1, 8, _BN), lambda i: (i, 0, 0)),
        ],
        out_shape=[
            jax.ShapeDtypeStruct((_N, _I), jnp.float32),
            jax.ShapeDtypeStruct((_N, 3), jnp.float32),
            jax.ShapeDtypeStruct((_N, 1), jnp.float32),
            jax.ShapeDtypeStruct((_NB, 8, _BN), jnp.float32),
        ],
        interpret=_INTERPRET,
    )(pc, mk, W1, b1, W2, b2, W3, b3)

    pmax, pmin = _segment_sc(aux)

    flow = pl.pallas_call(
        _stage2_body,
        grid=grid,
        in_specs=[
            pl.BlockSpec((1, 8, _BN), lambda i: (i, 0, 0)),
            pl.BlockSpec((_NW, _I, 3, 16), lambda i: (0, 0, 0, 0)),
            pl.BlockSpec((_NW, _I, 3, 16), lambda i: (0, 0, 0, 0)),
        ],
        out_specs=pl.BlockSpec((_BN, 3), lambda i: (i, 0)),
        out_shape=jax.ShapeDtypeStruct((_N, 3), jnp.float32),
        interpret=_INTERPRET,
    )(aux, pmax, pmin)

    return (flow.reshape(1, _N, 3), m.reshape(1, _N, _I),
            t.reshape(1, _N, 3), yaw.reshape(1, _N, 1))


# speedup vs baseline: 1.3930x; 1.0028x over previous
import functools

import jax
import jax.numpy as jnp
from jax import lax
from jax.experimental import pallas as pl
from jax.experimental.pallas import tpu as pltpu
from jax.experimental.pallas import tpu_sc as plsc

_N = 100000
_I = 10
_H = 128
_BN = 4000
_NB = _N // _BN
_NW = 32
_NEG = -3.0e38
_POS = 3.0e38
_INTERPRET = False


def _stage1_body(pc_ref, mk_ref, w1_ref, b1_ref, w2_ref, b2_ref, w3_ref, b3_ref,
                 m_ref, aux_ref):
    x = pc_ref[...]
    mk = mk_ref[...]

    h = jnp.maximum(
        jnp.dot(x, w1_ref[...], preferred_element_type=jnp.float32)
        + b1_ref[...][None, :], 0.0)
    h = jnp.maximum(
        jnp.dot(h, w2_ref[...], preferred_element_type=jnp.float32)
        + b2_ref[...][None, :], 0.0)
    out4 = (jnp.dot(h, w3_ref[...], preferred_element_type=jnp.float32)
            + b3_ref[...][None, :])

    mmax = jnp.max(mk, axis=1, keepdims=True)
    sh = mk - mmax
    e = jnp.exp(sh)
    s = jnp.sum(e, axis=1, keepdims=True)
    m_ref[...] = jnp.exp(sh - jnp.log(s))

    bn = x.shape[0]
    mkT = jnp.transpose(mk)
    pcT = jnp.transpose(x)
    out4T = jnp.transpose(out4)
    mmaxT = jnp.max(mkT, axis=0, keepdims=True)
    subi = jax.lax.broadcasted_iota(jnp.int32, (_I, bn), 0)
    big = jnp.int32(2 ** 30)
    indT = jnp.min(jnp.where(mkT == mmaxT, subi, big),
                   axis=0, keepdims=True)
    indTf = indT.astype(jnp.float32)
    aux_ref[0] = jnp.concatenate([out4T, pcT, indTf], axis=0)


def _seg_sc_body(aux_hbm, pmax_hbm, pmin_hbm, buf, omax, omin):
    cid = lax.axis_index("c")
    sid = lax.axis_index("s")
    w = sid * 2 + cid

    for inst in range(_I):
        for cc in range(3):
            omax[inst, cc, :] = jnp.full((16,), _NEG, jnp.float32)
            omin[inst, cc, :] = jnp.full((16,), _POS, jnp.float32)

    @pl.when(w < _NB)
    def _work():
        pltpu.sync_copy(aux_hbm.at[w], buf)

        for pair in range(_I // 2):
            i0 = 2 * pair
            i1 = i0 + 1

            def body(j, carry):
                (ax0, ay0, az0, nx0, ny0, nz0,
                 ax1, ay1, az1, nx1, ny1, nz1) = carry
                base = j * 16
                vx = buf[4, pl.ds(base, 16)]
                vy = buf[5, pl.ds(base, 16)]
                vz = buf[6, pl.ds(base, 16)]
                vi = buf[7, pl.ds(base, 16)]
                c0 = vi == float(i0)
                c1 = vi == float(i1)
                ax0 = jnp.maximum(ax0, jnp.where(c0, vx, _NEG))
                ay0 = jnp.maximum(ay0, jnp.where(c0, vy, _NEG))
                az0 = jnp.maximum(az0, jnp.where(c0, vz, _NEG))
                nx0 = jnp.minimum(nx0, jnp.where(c0, vx, _POS))
                ny0 = jnp.minimum(ny0, jnp.where(c0, vy, _POS))
                nz0 = jnp.minimum(nz0, jnp.where(c0, vz, _POS))
                ax1 = jnp.maximum(ax1, jnp.where(c1, vx, _NEG))
                ay1 = jnp.maximum(ay1, jnp.where(c1, vy, _NEG))
                az1 = jnp.maximum(az1, jnp.where(c1, vz, _NEG))
                nx1 = jnp.minimum(nx1, jnp.where(c1, vx, _POS))
                ny1 = jnp.minimum(ny1, jnp.where(c1, vy, _POS))
                nz1 = jnp.minimum(nz1, jnp.where(c1, vz, _POS))
                return (ax0, ay0, az0, nx0, ny0, nz0,
                        ax1, ay1, az1, nx1, ny1, nz1)

            neg = jnp.full((16,), _NEG, jnp.float32)
            pos = jnp.full((16,), _POS, jnp.float32)
            res = lax.fori_loop(
                0, _BN // 16, body,
                (neg, neg, neg, pos, pos, pos,
                 neg, neg, neg, pos, pos, pos))
            omax[i0, 0, :] = res[0]
            omax[i0, 1, :] = res[1]
            omax[i0, 2, :] = res[2]
            omin[i0, 0, :] = res[3]
            omin[i0, 1, :] = res[4]
            omin[i0, 2, :] = res[5]
            omax[i1, 0, :] = res[6]
            omax[i1, 1, :] = res[7]
            omax[i1, 2, :] = res[8]
            omin[i1, 0, :] = res[9]
            omin[i1, 1, :] = res[10]
            omin[i1, 2, :] = res[11]

    pltpu.sync_copy(omax, pmax_hbm.at[w])
    pltpu.sync_copy(omin, pmin_hbm.at[w])


def _segment_sc(aux):
    mesh = plsc.VectorSubcoreMesh(core_axis_name="c", subcore_axis_name="s")
    fn = functools.partial(
        pl.kernel, mesh=mesh,
        out_type=[
            jax.ShapeDtypeStruct((_NW, _I, 3, 16), jnp.float32),
            jax.ShapeDtypeStruct((_NW, _I, 3, 16), jnp.float32),
        ],
        scratch_types=[
            pltpu.VMEM((8, _BN), jnp.float32),
            pltpu.VMEM((_I, 3, 16), jnp.float32),
            pltpu.VMEM((_I, 3, 16), jnp.float32),
        ],
    )(_seg_sc_body)
    return fn(aux)


def _stage2_body(aux_ref, pmax_ref, pmin_ref, flow_ref, t_ref, yaw_ref):
    ax = aux_ref[0]
    z = ax[3:4, :]
    px = ax[4:5, :]
    py = ax[5:6, :]
    pz = ax[6:7, :]
    ind = ax[7:8, :]

    rmax = pmax_ref[0]
    rmin = pmin_ref[0]
    for wx in range(1, _NW):
        rmax = jnp.maximum(rmax, pmax_ref[wx])
        rmin = jnp.minimum(rmin, pmin_ref[wx])
    vmax = jnp.max(rmax, axis=2)
    vmin = jnp.min(rmin, axis=2)

    zero = jnp.zeros_like(z)
    accx, accy, accz = zero, zero, zero

    def _center(c, i):
        mx = vmax[i, c]
        mn = vmin[i, c]
        return jnp.where(mx >= mn, (mx + mn) * 0.5, 0.0)

    for i in range(_I):
        cond = ind == float(i)
        accx = jnp.where(cond, _center(0, i), accx)
        accy = jnp.where(cond, _center(1, i), accy)
        accz = jnp.where(cond, _center(2, i), accz)

    dx = px - accx
    dy = py - accy
    dz = pz - accz

    a2 = z * z
    a = jnp.sqrt(a2 + 1e-12)
    small = a < 1e-4
    a_safe = jnp.where(small, 1.0, a)
    sin_term = jnp.where(small, 1.0 - a2 / 6.0, jnp.sin(a_safe) / a_safe)
    cos_term = jnp.where(small, 0.5 - a2 / 24.0,
                         (1.0 - jnp.cos(a_safe)) / (a_safe * a_safe))
    s = sin_term * z
    cm = 1.0 - cos_term * a2

    rx = cm * dx - s * dy
    ry = s * dx + cm * dy
    fx = (rx + accx + ax[0:1, :]) - px
    fy = (ry + accy + ax[1:2, :]) - py
    fz = (dz + accz + ax[2:3, :]) - pz
    flowT = jnp.concatenate([fx, fy, fz], axis=0)
    flow_ref[...] = jnp.transpose(flowT)
    t_ref[...] = jnp.transpose(ax[0:3, :])
    yaw_ref[...] = jnp.transpose(ax[3:4, :])


def kernel(pc1, mask, W1, b1, W2, b2, W3, b3):
    pc = pc1.reshape(_N, 3)
    mk = mask.reshape(_N, _I)
    grid = (_NB,)

    m, aux = pl.pallas_call(
        _stage1_body,
        grid=grid,
        in_specs=[
            pl.BlockSpec((_BN, 3), lambda i: (i, 0)),
            pl.BlockSpec((_BN, _I), lambda i: (i, 0)),
            pl.BlockSpec((3, _H), lambda i: (0, 0)),
            pl.BlockSpec((_H,), lambda i: (0,)),
            pl.BlockSpec((_H, _H), lambda i: (0, 0)),
            pl.BlockSpec((_H,), lambda i: (0,)),
            pl.BlockSpec((_H, 4), lambda i: (0, 0)),
            pl.BlockSpec((4,), lambda i: (0,)),
        ],
        out_specs=[
            pl.BlockSpec((_BN, _I), lambda i: (i, 0)),
            pl.BlockSpec((1, 8, _BN), lambda i: (i, 0, 0)),
        ],
        out_shape=[
            jax.ShapeDtypeStruct((_N, _I), jnp.float32),
            jax.ShapeDtypeStruct((_NB, 8, _BN), jnp.float32),
        ],
        interpret=_INTERPRET,
    )(pc, mk, W1, b1, W2, b2, W3, b3)

    pmax, pmin = _segment_sc(aux)

    flow, t, yaw = pl.pallas_call(
        _stage2_body,
        grid=grid,
        in_specs=[
            pl.BlockSpec((1, 8, _BN), lambda i: (i, 0, 0)),
            pl.BlockSpec((_NW, _I, 3, 16), lambda i: (0, 0, 0, 0)),
            pl.BlockSpec((_NW, _I, 3, 16), lambda i: (0, 0, 0, 0)),
        ],
        out_specs=[
            pl.BlockSpec((_BN, 3), lambda i: (i, 0)),
            pl.BlockSpec((_BN, 3), lambda i: (i, 0)),
            pl.BlockSpec((_BN, 1), lambda i: (i, 0)),
        ],
        out_shape=[
            jax.ShapeDtypeStruct((_N, 3), jnp.float32),
            jax.ShapeDtypeStruct((_N, 3), jnp.float32),
            jax.ShapeDtypeStruct((_N, 1), jnp.float32),
        ],
        interpret=_INTERPRET,
    )(aux, pmax, pmin)

    return (flow.reshape(1, _N, 3), m.reshape(1, _N, _I),
            t.reshape(1, _N, 3), yaw.reshape(1, _N, 1))


# speedup vs baseline: 1.5285x; 1.0972x over previous
import functools

import jax
import jax.numpy as jnp
from jax import lax
from jax.experimental import pallas as pl
from jax.experimental.pallas import tpu as pltpu
from jax.experimental.pallas import tpu_sc as plsc

_N = 100000
_I = 10
_H = 128
_BN = 4000
_NB = _N // _BN
_NW = 32
_NEG = -3.0e38
_POS = 3.0e38
_INTERPRET = False


def _stage1_body(pc_ref, mkt_ref, w1_ref, b1_ref, w2_ref, b2_ref, w3_ref, b3_ref,
                 mt_ref, aux_ref):
    x = pc_ref[...]
    mkT = mkt_ref[0]

    h = jnp.maximum(
        jnp.dot(x, w1_ref[...], preferred_element_type=jnp.float32)
        + b1_ref[...][None, :], 0.0)
    h = jnp.maximum(
        jnp.dot(h, w2_ref[...], preferred_element_type=jnp.float32)
        + b2_ref[...][None, :], 0.0)
    out4 = (jnp.dot(h, w3_ref[...], preferred_element_type=jnp.float32)
            + b3_ref[...][None, :])

    bn = x.shape[0]
    pcT = jnp.transpose(x)
    out4T = jnp.transpose(out4)
    mmaxT = jnp.max(mkT, axis=0, keepdims=True)
    shT = mkT - mmaxT
    eT = jnp.exp(shT)
    sT = jnp.sum(eT, axis=0, keepdims=True)
    mt_ref[0] = jnp.exp(shT - jnp.log(sT))
    subi = jax.lax.broadcasted_iota(jnp.int32, (_I, bn), 0)
    big = jnp.int32(2 ** 30)
    indT = jnp.min(jnp.where(mkT == mmaxT, subi, big),
                   axis=0, keepdims=True)
    indTf = indT.astype(jnp.float32)
    aux_ref[0] = jnp.concatenate([out4T, pcT, indTf], axis=0)


def _seg_sc_body(aux_hbm, pmax_hbm, pmin_hbm, buf, omax, omin):
    cid = lax.axis_index("c")
    sid = lax.axis_index("s")
    w = sid * 2 + cid

    for inst in range(_I):
        for cc in range(3):
            omax[inst, cc, :] = jnp.full((16,), _NEG, jnp.float32)
            omin[inst, cc, :] = jnp.full((16,), _POS, jnp.float32)

    @pl.when(w < _NB)
    def _work():
        pltpu.sync_copy(aux_hbm.at[w], buf)

        for pair in range(_I // 2):
            i0 = 2 * pair
            i1 = i0 + 1

            def body(j, carry):
                (ax0, ay0, az0, nx0, ny0, nz0,
                 ax1, ay1, az1, nx1, ny1, nz1) = carry
                base = j * 16
                vx = buf[4, pl.ds(base, 16)]
                vy = buf[5, pl.ds(base, 16)]
                vz = buf[6, pl.ds(base, 16)]
                vi = buf[7, pl.ds(base, 16)]
                c0 = vi == float(i0)
                c1 = vi == float(i1)
                ax0 = jnp.maximum(ax0, jnp.where(c0, vx, _NEG))
                ay0 = jnp.maximum(ay0, jnp.where(c0, vy, _NEG))
                az0 = jnp.maximum(az0, jnp.where(c0, vz, _NEG))
                nx0 = jnp.minimum(nx0, jnp.where(c0, vx, _POS))
                ny0 = jnp.minimum(ny0, jnp.where(c0, vy, _POS))
                nz0 = jnp.minimum(nz0, jnp.where(c0, vz, _POS))
                ax1 = jnp.maximum(ax1, jnp.where(c1, vx, _NEG))
                ay1 = jnp.maximum(ay1, jnp.where(c1, vy, _NEG))
                az1 = jnp.maximum(az1, jnp.where(c1, vz, _NEG))
                nx1 = jnp.minimum(nx1, jnp.where(c1, vx, _POS))
                ny1 = jnp.minimum(ny1, jnp.where(c1, vy, _POS))
                nz1 = jnp.minimum(nz1, jnp.where(c1, vz, _POS))
                return (ax0, ay0, az0, nx0, ny0, nz0,
                        ax1, ay1, az1, nx1, ny1, nz1)

            neg = jnp.full((16,), _NEG, jnp.float32)
            pos = jnp.full((16,), _POS, jnp.float32)
            res = lax.fori_loop(
                0, _BN // 16, body,
                (neg, neg, neg, pos, pos, pos,
                 neg, neg, neg, pos, pos, pos))
            omax[i0, 0, :] = res[0]
            omax[i0, 1, :] = res[1]
            omax[i0, 2, :] = res[2]
            omin[i0, 0, :] = res[3]
            omin[i0, 1, :] = res[4]
            omin[i0, 2, :] = res[5]
            omax[i1, 0, :] = res[6]
            omax[i1, 1, :] = res[7]
            omax[i1, 2, :] = res[8]
            omin[i1, 0, :] = res[9]
            omin[i1, 1, :] = res[10]
            omin[i1, 2, :] = res[11]

    pltpu.sync_copy(omax, pmax_hbm.at[w])
    pltpu.sync_copy(omin, pmin_hbm.at[w])


def _segment_sc(aux):
    mesh = plsc.VectorSubcoreMesh(core_axis_name="c", subcore_axis_name="s")
    fn = functools.partial(
        pl.kernel, mesh=mesh,
        out_type=[
            jax.ShapeDtypeStruct((_NW, _I, 3, 16), jnp.float32),
            jax.ShapeDtypeStruct((_NW, _I, 3, 16), jnp.float32),
        ],
        scratch_types=[
            pltpu.VMEM((8, _BN), jnp.float32),
            pltpu.VMEM((_I, 3, 16), jnp.float32),
            pltpu.VMEM((_I, 3, 16), jnp.float32),
        ],
    )(_seg_sc_body)
    return fn(aux)


def _stage2_body(aux_ref, pmax_ref, pmin_ref, flow_ref, t_ref, yaw_ref):
    ax = aux_ref[0]
    z = ax[3:4, :]
    px = ax[4:5, :]
    py = ax[5:6, :]
    pz = ax[6:7, :]
    ind = ax[7:8, :]

    rmax = pmax_ref[0]
    rmin = pmin_ref[0]
    for wx in range(1, _NW):
        rmax = jnp.maximum(rmax, pmax_ref[wx])
        rmin = jnp.minimum(rmin, pmin_ref[wx])
    vmax = jnp.max(rmax, axis=2)
    vmin = jnp.min(rmin, axis=2)

    zero = jnp.zeros_like(z)
    accx, accy, accz = zero, zero, zero

    def _center(c, i):
        mx = vmax[i, c]
        mn = vmin[i, c]
        return jnp.where(mx >= mn, (mx + mn) * 0.5, 0.0)

    for i in range(_I):
        cond = ind == float(i)
        accx = jnp.where(cond, _center(0, i), accx)
        accy = jnp.where(cond, _center(1, i), accy)
        accz = jnp.where(cond, _center(2, i), accz)

    dx = px - accx
    dy = py - accy
    dz = pz - accz

    a2 = z * z
    a = jnp.sqrt(a2 + 1e-12)
    small = a < 1e-4
    a_safe = jnp.where(small, 1.0, a)
    sin_term = jnp.where(small, 1.0 - a2 / 6.0, jnp.sin(a_safe) / a_safe)
    cos_term = jnp.where(small, 0.5 - a2 / 24.0,
                         (1.0 - jnp.cos(a_safe)) / (a_safe * a_safe))
    s = sin_term * z
    cm = 1.0 - cos_term * a2

    rx = cm * dx - s * dy
    ry = s * dx + cm * dy
    fx = (rx + accx + ax[0:1, :]) - px
    fy = (ry + accy + ax[1:2, :]) - py
    fz = (dz + accz + ax[2:3, :]) - pz
    flowT = jnp.concatenate([fx, fy, fz], axis=0)
    flow_ref[...] = jnp.transpose(flowT)
    t_ref[...] = jnp.transpose(ax[0:3, :])
    yaw_ref[...] = jnp.transpose(ax[3:4, :])


def kernel(pc1, mask, W1, b1, W2, b2, W3, b3):
    pc = pc1.reshape(_N, 3)
    mkT = jnp.transpose(mask.reshape(_NB, _BN, _I), (0, 2, 1))
    grid = (_NB,)

    mT, aux = pl.pallas_call(
        _stage1_body,
        grid=grid,
        in_specs=[
            pl.BlockSpec((_BN, 3), lambda i: (i, 0)),
            pl.BlockSpec((1, _I, _BN), lambda i: (i, 0, 0)),
            pl.BlockSpec((3, _H), lambda i: (0, 0)),
            pl.BlockSpec((_H,), lambda i: (0,)),
            pl.BlockSpec((_H, _H), lambda i: (0, 0)),
            pl.BlockSpec((_H,), lambda i: (0,)),
            pl.BlockSpec((_H, 4), lambda i: (0, 0)),
            pl.BlockSpec((4,), lambda i: (0,)),
        ],
        out_specs=[
            pl.BlockSpec((1, _I, _BN), lambda i: (i, 0, 0)),
            pl.BlockSpec((1, 8, _BN), lambda i: (i, 0, 0)),
        ],
        out_shape=[
            jax.ShapeDtypeStruct((_NB, _I, _BN), jnp.float32),
            jax.ShapeDtypeStruct((_NB, 8, _BN), jnp.float32),
        ],
        interpret=_INTERPRET,
    )(pc, mkT, W1, b1, W2, b2, W3, b3)

    pmax, pmin = _segment_sc(aux)

    flow, t, yaw = pl.pallas_call(
        _stage2_body,
        grid=grid,
        in_specs=[
            pl.BlockSpec((1, 8, _BN), lambda i: (i, 0, 0)),
            pl.BlockSpec((_NW, _I, 3, 16), lambda i: (0, 0, 0, 0)),
            pl.BlockSpec((_NW, _I, 3, 16), lambda i: (0, 0, 0, 0)),
        ],
        out_specs=[
            pl.BlockSpec((_BN, 3), lambda i: (i, 0)),
            pl.BlockSpec((_BN, 3), lambda i: (i, 0)),
            pl.BlockSpec((_BN, 1), lambda i: (i, 0)),
        ],
        out_shape=[
            jax.ShapeDtypeStruct((_N, 3), jnp.float32),
            jax.ShapeDtypeStruct((_N, 3), jnp.float32),
            jax.ShapeDtypeStruct((_N, 1), jnp.float32),
        ],
        interpret=_INTERPRET,
    )(aux, pmax, pmin)

    m = jnp.transpose(mT, (0, 2, 1))
    return (flow.reshape(1, _N, 3), m.reshape(1, _N, _I),
            t.reshape(1, _N, 3), yaw.reshape(1, _N, 1))


# speedup vs baseline: 2.6419x; 1.7285x over previous
import functools

import jax
import jax.numpy as jnp
from jax import lax
from jax.experimental import pallas as pl
from jax.experimental.pallas import tpu as pltpu
from jax.experimental.pallas import tpu_sc as plsc

_N = 100000
_I = 10
_H = 128
_BN = 4000
_NB = _N // _BN
_NW = 32
_NEG = -3.0e38
_POS = 3.0e38
_INTERPRET = False


def _stage1_body(pct_ref, mkt_ref, w1t_ref, b1_ref, w2t_ref, b2_ref, w3t_ref, b3_ref,
                 mt_ref, aux_ref):
    pcT = pct_ref[0]
    mkT = mkt_ref[0]

    hT = jnp.maximum(
        jnp.dot(w1t_ref[...], pcT, preferred_element_type=jnp.float32)
        + b1_ref[...][:, None], 0.0)
    hT = jnp.maximum(
        jnp.dot(w2t_ref[...], hT, preferred_element_type=jnp.float32)
        + b2_ref[...][:, None], 0.0)
    out4T = (jnp.dot(w3t_ref[...], hT, preferred_element_type=jnp.float32)
             + b3_ref[...][:, None])

    bn = pcT.shape[1]
    mmaxT = jnp.max(mkT, axis=0, keepdims=True)
    shT = mkT - mmaxT
    eT = jnp.exp(shT)
    sT = jnp.sum(eT, axis=0, keepdims=True)
    mt_ref[0] = jnp.exp(shT - jnp.log(sT))
    subi = jax.lax.broadcasted_iota(jnp.int32, (_I, bn), 0)
    big = jnp.int32(2 ** 30)
    indT = jnp.min(jnp.where(mkT == mmaxT, subi, big),
                   axis=0, keepdims=True)
    indTf = indT.astype(jnp.float32)
    aux_ref[0] = jnp.concatenate([out4T, pcT, indTf], axis=0)


def _seg_sc_body(aux_hbm, pmax_hbm, pmin_hbm, buf, omax, omin):
    cid = lax.axis_index("c")
    sid = lax.axis_index("s")
    w = sid * 2 + cid

    for inst in range(_I):
        for cc in range(3):
            omax[inst, cc, :] = jnp.full((16,), _NEG, jnp.float32)
            omin[inst, cc, :] = jnp.full((16,), _POS, jnp.float32)

    @pl.when(w < _NB)
    def _work():
        pltpu.sync_copy(aux_hbm.at[w], buf)

        for pair in range(_I // 2):
            i0 = 2 * pair
            i1 = i0 + 1

            def body(j, carry):
                (ax0, ay0, az0, nx0, ny0, nz0,
                 ax1, ay1, az1, nx1, ny1, nz1) = carry
                base = j * 16
                vx = buf[4, pl.ds(base, 16)]
                vy = buf[5, pl.ds(base, 16)]
                vz = buf[6, pl.ds(base, 16)]
                vi = buf[7, pl.ds(base, 16)]
                c0 = vi == float(i0)
                c1 = vi == float(i1)
                ax0 = jnp.maximum(ax0, jnp.where(c0, vx, _NEG))
                ay0 = jnp.maximum(ay0, jnp.where(c0, vy, _NEG))
                az0 = jnp.maximum(az0, jnp.where(c0, vz, _NEG))
                nx0 = jnp.minimum(nx0, jnp.where(c0, vx, _POS))
                ny0 = jnp.minimum(ny0, jnp.where(c0, vy, _POS))
                nz0 = jnp.minimum(nz0, jnp.where(c0, vz, _POS))
                ax1 = jnp.maximum(ax1, jnp.where(c1, vx, _NEG))
                ay1 = jnp.maximum(ay1, jnp.where(c1, vy, _NEG))
                az1 = jnp.maximum(az1, jnp.where(c1, vz, _NEG))
                nx1 = jnp.minimum(nx1, jnp.where(c1, vx, _POS))
                ny1 = jnp.minimum(ny1, jnp.where(c1, vy, _POS))
                nz1 = jnp.minimum(nz1, jnp.where(c1, vz, _POS))
                return (ax0, ay0, az0, nx0, ny0, nz0,
                        ax1, ay1, az1, nx1, ny1, nz1)

            neg = jnp.full((16,), _NEG, jnp.float32)
            pos = jnp.full((16,), _POS, jnp.float32)
            res = lax.fori_loop(
                0, _BN // 16, body,
                (neg, neg, neg, pos, pos, pos,
                 neg, neg, neg, pos, pos, pos))
            omax[i0, 0, :] = res[0]
            omax[i0, 1, :] = res[1]
            omax[i0, 2, :] = res[2]
            omin[i0, 0, :] = res[3]
            omin[i0, 1, :] = res[4]
            omin[i0, 2, :] = res[5]
            omax[i1, 0, :] = res[6]
            omax[i1, 1, :] = res[7]
            omax[i1, 2, :] = res[8]
            omin[i1, 0, :] = res[9]
            omin[i1, 1, :] = res[10]
            omin[i1, 2, :] = res[11]

    pltpu.sync_copy(omax, pmax_hbm.at[w])
    pltpu.sync_copy(omin, pmin_hbm.at[w])


def _segment_sc(aux):
    mesh = plsc.VectorSubcoreMesh(core_axis_name="c", subcore_axis_name="s")
    fn = functools.partial(
        pl.kernel, mesh=mesh,
        out_type=[
            jax.ShapeDtypeStruct((_NW, _I, 3, 16), jnp.float32),
            jax.ShapeDtypeStruct((_NW, _I, 3, 16), jnp.float32),
        ],
        scratch_types=[
            pltpu.VMEM((8, _BN), jnp.float32),
            pltpu.VMEM((_I, 3, 16), jnp.float32),
            pltpu.VMEM((_I, 3, 16), jnp.float32),
        ],
    )(_seg_sc_body)
    return fn(aux)


def _stage2_body(aux_ref, pmax_ref, pmin_ref, flow_ref, t_ref, yaw_ref):
    ax = aux_ref[0]
    z = ax[3:4, :]
    px = ax[4:5, :]
    py = ax[5:6, :]
    pz = ax[6:7, :]
    ind = ax[7:8, :]

    rmax = pmax_ref[0]
    rmin = pmin_ref[0]
    for wx in range(1, _NW):
        rmax = jnp.maximum(rmax, pmax_ref[wx])
        rmin = jnp.minimum(rmin, pmin_ref[wx])
    vmax = jnp.max(rmax, axis=2)
    vmin = jnp.min(rmin, axis=2)

    zero = jnp.zeros_like(z)
    accx, accy, accz = zero, zero, zero

    def _center(c, i):
        mx = vmax[i, c]
        mn = vmin[i, c]
        return jnp.where(mx >= mn, (mx + mn) * 0.5, 0.0)

    for i in range(_I):
        cond = ind == float(i)
        accx = jnp.where(cond, _center(0, i), accx)
        accy = jnp.where(cond, _center(1, i), accy)
        accz = jnp.where(cond, _center(2, i), accz)

    dx = px - accx
    dy = py - accy
    dz = pz - accz

    a2 = z * z
    a = jnp.sqrt(a2 + 1e-12)
    small = a < 1e-4
    a_safe = jnp.where(small, 1.0, a)
    sin_term = jnp.where(small, 1.0 - a2 / 6.0, jnp.sin(a_safe) / a_safe)
    cos_term = jnp.where(small, 0.5 - a2 / 24.0,
                         (1.0 - jnp.cos(a_safe)) / (a_safe * a_safe))
    s = sin_term * z
    cm = 1.0 - cos_term * a2

    rx = cm * dx - s * dy
    ry = s * dx + cm * dy
    fx = (rx + accx + ax[0:1, :]) - px
    fy = (ry + accy + ax[1:2, :]) - py
    fz = (dz + accz + ax[2:3, :]) - pz
    flowT = jnp.concatenate([fx, fy, fz], axis=0)
    flow_ref[0] = flowT
    t_ref[0] = ax[0:3, :]
    yaw_ref[0] = ax[3:4, :]


def kernel(pc1, mask, W1, b1, W2, b2, W3, b3):
    pcT = jnp.transpose(pc1.reshape(_NB, _BN, 3), (0, 2, 1))
    mkT = jnp.transpose(mask.reshape(_NB, _BN, _I), (0, 2, 1))
    W1T, W2T, W3T = W1.T, W2.T, W3.T
    grid = (_NB,)

    mT, aux = pl.pallas_call(
        _stage1_body,
        grid=grid,
        in_specs=[
            pl.BlockSpec((1, 3, _BN), lambda i: (i, 0, 0)),
            pl.BlockSpec((1, _I, _BN), lambda i: (i, 0, 0)),
            pl.BlockSpec((_H, 3), lambda i: (0, 0)),
            pl.BlockSpec((_H,), lambda i: (0,)),
            pl.BlockSpec((_H, _H), lambda i: (0, 0)),
            pl.BlockSpec((_H,), lambda i: (0,)),
            pl.BlockSpec((4, _H), lambda i: (0, 0)),
            pl.BlockSpec((4,), lambda i: (0,)),
        ],
        out_specs=[
            pl.BlockSpec((1, _I, _BN), lambda i: (i, 0, 0)),
            pl.BlockSpec((1, 8, _BN), lambda i: (i, 0, 0)),
        ],
        out_shape=[
            jax.ShapeDtypeStruct((_NB, _I, _BN), jnp.float32),
            jax.ShapeDtypeStruct((_NB, 8, _BN), jnp.float32),
        ],
        interpret=_INTERPRET,
    )(pcT, mkT, W1T, b1, W2T, b2, W3T, b3)

    pmax, pmin = _segment_sc(aux)

    flow, t, yaw = pl.pallas_call(
        _stage2_body,
        grid=grid,
        in_specs=[
            pl.BlockSpec((1, 8, _BN), lambda i: (i, 0, 0)),
            pl.BlockSpec((_NW, _I, 3, 16), lambda i: (0, 0, 0, 0)),
            pl.BlockSpec((_NW, _I, 3, 16), lambda i: (0, 0, 0, 0)),
        ],
        out_specs=[
            pl.BlockSpec((1, 3, _BN), lambda i: (i, 0, 0)),
            pl.BlockSpec((1, 3, _BN), lambda i: (i, 0, 0)),
            pl.BlockSpec((1, 1, _BN), lambda i: (i, 0, 0)),
        ],
        out_shape=[
            jax.ShapeDtypeStruct((_NB, 3, _BN), jnp.float32),
            jax.ShapeDtypeStruct((_NB, 3, _BN), jnp.float32),
            jax.ShapeDtypeStruct((_NB, 1, _BN), jnp.float32),
        ],
        interpret=_INTERPRET,
    )(aux, pmax, pmin)

    m = jnp.transpose(mT, (0, 2, 1)).reshape(1, _N, _I)
    flow = jnp.transpose(flow, (0, 2, 1)).reshape(1, _N, 3)
    t = jnp.transpose(t, (0, 2, 1)).reshape(1, _N, 3)
    yaw = jnp.transpose(yaw, (0, 2, 1)).reshape(1, _N, 1)
    return (flow, m, t, yaw)


# speedup vs baseline: 2.6771x; 1.0133x over previous
import functools

import jax
import jax.numpy as jnp
from jax import lax
from jax.experimental import pallas as pl
from jax.experimental.pallas import tpu as pltpu
from jax.experimental.pallas import tpu_sc as plsc

_N = 100000
_I = 10
_H = 128
_BN = 4000
_NB = _N // _BN
_NW = 32
_NEG = -3.0e38
_POS = 3.0e38
_INTERPRET = False


def _stage1_body(pct_ref, mkt_ref, w1t_ref, b1_ref, w2t_ref, b2_ref, w3t_ref, b3_ref,
                 mt_ref, aux_ref):
    pcT = pct_ref[0]
    mkT = mkt_ref[0]

    hT = jnp.maximum(
        jnp.dot(w1t_ref[...], pcT, preferred_element_type=jnp.float32)
        + b1_ref[...][:, None], 0.0)
    hT = jnp.maximum(
        jnp.dot(w2t_ref[...].astype(jnp.bfloat16), hT.astype(jnp.bfloat16),
                preferred_element_type=jnp.float32)
        + b2_ref[...][:, None], 0.0)
    out4T = (jnp.dot(w3t_ref[...], hT, preferred_element_type=jnp.float32)
             + b3_ref[...][:, None])

    bn = pcT.shape[1]
    mmaxT = jnp.max(mkT, axis=0, keepdims=True)
    shT = mkT - mmaxT
    eT = jnp.exp(shT)
    sT = jnp.sum(eT, axis=0, keepdims=True)
    mt_ref[0] = jnp.exp(shT - jnp.log(sT))
    subi = jax.lax.broadcasted_iota(jnp.int32, (_I, bn), 0)
    big = jnp.int32(2 ** 30)
    indT = jnp.min(jnp.where(mkT == mmaxT, subi, big),
                   axis=0, keepdims=True)
    indTf = indT.astype(jnp.float32)
    aux_ref[0] = jnp.concatenate([out4T, pcT, indTf], axis=0)


def _seg_sc_body(aux_hbm, pmax_hbm, pmin_hbm, buf, omax, omin):
    cid = lax.axis_index("c")
    sid = lax.axis_index("s")
    w = sid * 2 + cid

    for inst in range(_I):
        for cc in range(3):
            omax[inst, cc, :] = jnp.full((16,), _NEG, jnp.float32)
            omin[inst, cc, :] = jnp.full((16,), _POS, jnp.float32)

    @pl.when(w < _NB)
    def _work():
        pltpu.sync_copy(aux_hbm.at[w], buf)

        for pair in range(_I // 2):
            i0 = 2 * pair
            i1 = i0 + 1

            def body(j, carry):
                (ax0, ay0, az0, nx0, ny0, nz0,
                 ax1, ay1, az1, nx1, ny1, nz1) = carry
                base = j * 16
                vx = buf[4, pl.ds(base, 16)]
                vy = buf[5, pl.ds(base, 16)]
                vz = buf[6, pl.ds(base, 16)]
                vi = buf[7, pl.ds(base, 16)]
                c0 = vi == float(i0)
                c1 = vi == float(i1)
                ax0 = jnp.maximum(ax0, jnp.where(c0, vx, _NEG))
                ay0 = jnp.maximum(ay0, jnp.where(c0, vy, _NEG))
                az0 = jnp.maximum(az0, jnp.where(c0, vz, _NEG))
                nx0 = jnp.minimum(nx0, jnp.where(c0, vx, _POS))
                ny0 = jnp.minimum(ny0, jnp.where(c0, vy, _POS))
                nz0 = jnp.minimum(nz0, jnp.where(c0, vz, _POS))
                ax1 = jnp.maximum(ax1, jnp.where(c1, vx, _NEG))
                ay1 = jnp.maximum(ay1, jnp.where(c1, vy, _NEG))
                az1 = jnp.maximum(az1, jnp.where(c1, vz, _NEG))
                nx1 = jnp.minimum(nx1, jnp.where(c1, vx, _POS))
                ny1 = jnp.minimum(ny1, jnp.where(c1, vy, _POS))
                nz1 = jnp.minimum(nz1, jnp.where(c1, vz, _POS))
                return (ax0, ay0, az0, nx0, ny0, nz0,
                        ax1, ay1, az1, nx1, ny1, nz1)

            neg = jnp.full((16,), _NEG, jnp.float32)
            pos = jnp.full((16,), _POS, jnp.float32)
            res = lax.fori_loop(
                0, _BN // 16, body,
                (neg, neg, neg, pos, pos, pos,
                 neg, neg, neg, pos, pos, pos))
            omax[i0, 0, :] = res[0]
            omax[i0, 1, :] = res[1]
            omax[i0, 2, :] = res[2]
            omin[i0, 0, :] = res[3]
            omin[i0, 1, :] = res[4]
            omin[i0, 2, :] = res[5]
            omax[i1, 0, :] = res[6]
            omax[i1, 1, :] = res[7]
            omax[i1, 2, :] = res[8]
            omin[i1, 0, :] = res[9]
            omin[i1, 1, :] = res[10]
            omin[i1, 2, :] = res[11]

    pltpu.sync_copy(omax, pmax_hbm.at[w])
    pltpu.sync_copy(omin, pmin_hbm.at[w])


def _segment_sc(aux):
    mesh = plsc.VectorSubcoreMesh(core_axis_name="c", subcore_axis_name="s")
    fn = functools.partial(
        pl.kernel, mesh=mesh,
        out_type=[
            jax.ShapeDtypeStruct((_NW, _I, 3, 16), jnp.float32),
            jax.ShapeDtypeStruct((_NW, _I, 3, 16), jnp.float32),
        ],
        scratch_types=[
            pltpu.VMEM((8, _BN), jnp.float32),
            pltpu.VMEM((_I, 3, 16), jnp.float32),
            pltpu.VMEM((_I, 3, 16), jnp.float32),
        ],
    )(_seg_sc_body)
    return fn(aux)


def _stage2_body(aux_ref, pmax_ref, pmin_ref, flow_ref, t_ref, yaw_ref,
                 vmaxs_ref, vmins_ref):
    ax = aux_ref[0]
    z = ax[3:4, :]
    px = ax[4:5, :]
    py = ax[5:6, :]
    pz = ax[6:7, :]
    ind = ax[7:8, :]

    @pl.when(pl.program_id(0) == 0)
    def _fold():
        rmax = pmax_ref[0]
        rmin = pmin_ref[0]
        for wx in range(1, _NW):
            rmax = jnp.maximum(rmax, pmax_ref[wx])
            rmin = jnp.minimum(rmin, pmin_ref[wx])
        vmaxs_ref[...] = jnp.max(rmax, axis=2)
        vmins_ref[...] = jnp.min(rmin, axis=2)

    zero = jnp.zeros_like(z)
    accx, accy, accz = zero, zero, zero

    def _center(c, i):
        mx = vmaxs_ref[i, c]
        mn = vmins_ref[i, c]
        return jnp.where(mx >= mn, (mx + mn) * 0.5, 0.0)

    for i in range(_I):
        cond = ind == float(i)
        accx = jnp.where(cond, _center(0, i), accx)
        accy = jnp.where(cond, _center(1, i), accy)
        accz = jnp.where(cond, _center(2, i), accz)

    dx = px - accx
    dy = py - accy
    dz = pz - accz

    a2 = z * z
    a = jnp.sqrt(a2 + 1e-12)
    small = a < 1e-4
    a_safe = jnp.where(small, 1.0, a)
    sin_term = jnp.where(small, 1.0 - a2 / 6.0, jnp.sin(a_safe) / a_safe)
    cos_term = jnp.where(small, 0.5 - a2 / 24.0,
                         (1.0 - jnp.cos(a_safe)) / (a_safe * a_safe))
    s = sin_term * z
    cm = 1.0 - cos_term * a2

    rx = cm * dx - s * dy
    ry = s * dx + cm * dy
    fx = (rx + accx + ax[0:1, :]) - px
    fy = (ry + accy + ax[1:2, :]) - py
    fz = (dz + accz + ax[2:3, :]) - pz
    flowT = jnp.concatenate([fx, fy, fz], axis=0)
    flow_ref[0] = flowT
    t_ref[0] = ax[0:3, :]
    yaw_ref[0] = ax[3:4, :]


def kernel(pc1, mask, W1, b1, W2, b2, W3, b3):
    pcT = jnp.transpose(pc1.reshape(_NB, _BN, 3), (0, 2, 1))
    mkT = jnp.transpose(mask.reshape(_NB, _BN, _I), (0, 2, 1))
    W1T, W2T, W3T = W1.T, W2.T, W3.T
    grid = (_NB,)

    mT, aux = pl.pallas_call(
        _stage1_body,
        grid=grid,
        in_specs=[
            pl.BlockSpec((1, 3, _BN), lambda i: (i, 0, 0)),
            pl.BlockSpec((1, _I, _BN), lambda i: (i, 0, 0)),
            pl.BlockSpec((_H, 3), lambda i: (0, 0)),
            pl.BlockSpec((_H,), lambda i: (0,)),
            pl.BlockSpec((_H, _H), lambda i: (0, 0)),
            pl.BlockSpec((_H,), lambda i: (0,)),
            pl.BlockSpec((4, _H), lambda i: (0, 0)),
            pl.BlockSpec((4,), lambda i: (0,)),
        ],
        out_specs=[
            pl.BlockSpec((1, _I, _BN), lambda i: (i, 0, 0)),
            pl.BlockSpec((1, 8, _BN), lambda i: (i, 0, 0)),
        ],
        out_shape=[
            jax.ShapeDtypeStruct((_NB, _I, _BN), jnp.float32),
            jax.ShapeDtypeStruct((_NB, 8, _BN), jnp.float32),
        ],
        interpret=_INTERPRET,
    )(pcT, mkT, W1T, b1, W2T, b2, W3T, b3)

    pmax, pmin = _segment_sc(aux)

    flow, t, yaw = pl.pallas_call(
        _stage2_body,
        grid=grid,
        in_specs=[
            pl.BlockSpec((1, 8, _BN), lambda i: (i, 0, 0)),
            pl.BlockSpec((_NW, _I, 3, 16), lambda i: (0, 0, 0, 0)),
            pl.BlockSpec((_NW, _I, 3, 16), lambda i: (0, 0, 0, 0)),
        ],
        out_specs=[
            pl.BlockSpec((1, 3, _BN), lambda i: (i, 0, 0)),
            pl.BlockSpec((1, 3, _BN), lambda i: (i, 0, 0)),
            pl.BlockSpec((1, 1, _BN), lambda i: (i, 0, 0)),
        ],
        out_shape=[
            jax.ShapeDtypeStruct((_NB, 3, _BN), jnp.float32),
            jax.ShapeDtypeStruct((_NB, 3, _BN), jnp.float32),
            jax.ShapeDtypeStruct((_NB, 1, _BN), jnp.float32),
        ],
        scratch_shapes=[
            pltpu.VMEM((_I, 3), jnp.float32),
            pltpu.VMEM((_I, 3), jnp.float32),
        ],
        interpret=_INTERPRET,
    )(aux, pmax, pmin)

    m = jnp.transpose(mT, (0, 2, 1)).reshape(1, _N, _I)
    flow = jnp.transpose(flow, (0, 2, 1)).reshape(1, _N, 3)
    t = jnp.transpose(t, (0, 2, 1)).reshape(1, _N, 3)
    yaw = jnp.transpose(yaw, (0, 2, 1)).reshape(1, _N, 1)
    return (flow, m, t, yaw)


# speedup vs baseline: 2.6779x; 1.0003x over previous
import functools

import jax
import jax.numpy as jnp
from jax import lax
from jax.experimental import pallas as pl
from jax.experimental.pallas import tpu as pltpu
from jax.experimental.pallas import tpu_sc as plsc

_N = 100000
_I = 10
_H = 128
_BN = 4000
_NB = _N // _BN
_NW = 32
_NEG = -3.0e38
_POS = 3.0e38


def _stage1_body(pct_ref, mkt_ref, w1t_ref, b1_ref, w2t_ref, b2_ref, w3t_ref, b3_ref,
                 mt_ref, aux_ref):
    pcT = pct_ref[0]
    mkT = mkt_ref[0]

    hT = jnp.maximum(
        jnp.dot(w1t_ref[...], pcT, preferred_element_type=jnp.float32)
        + b1_ref[...][:, None], 0.0)
    hT = jnp.maximum(
        jnp.dot(w2t_ref[...].astype(jnp.bfloat16), hT.astype(jnp.bfloat16),
                preferred_element_type=jnp.float32)
        + b2_ref[...][:, None], 0.0)
    out4T = (jnp.dot(w3t_ref[...], hT, preferred_element_type=jnp.float32)
             + b3_ref[...][:, None])

    bn = pcT.shape[1]
    mmaxT = jnp.max(mkT, axis=0, keepdims=True)
    shT = mkT - mmaxT
    eT = jnp.exp(shT)
    sT = jnp.sum(eT, axis=0, keepdims=True)
    mt_ref[0] = jnp.exp(shT - jnp.log(sT))
    subi = jax.lax.broadcasted_iota(jnp.int32, (_I, bn), 0)
    big = jnp.int32(2 ** 30)
    indT = jnp.min(jnp.where(mkT == mmaxT, subi, big),
                   axis=0, keepdims=True)
    indTf = indT.astype(jnp.float32)
    aux_ref[0] = jnp.concatenate([out4T, pcT, indTf], axis=0)


def _seg_sc_body(aux_hbm, pmax_hbm, pmin_hbm, buf, omax, omin):
    cid = lax.axis_index("c")
    sid = lax.axis_index("s")
    w = sid * 2 + cid

    for inst in range(_I):
        for cc in range(3):
            omax[inst, cc, :] = jnp.full((16,), _NEG, jnp.float32)
            omin[inst, cc, :] = jnp.full((16,), _POS, jnp.float32)

    @pl.when(w < _NB)
    def _work():
        pltpu.sync_copy(aux_hbm.at[w], buf)

        for pair in range(_I // 2):
            i0 = 2 * pair
            i1 = i0 + 1

            def body(j, carry):
                (ax0, ay0, az0, nx0, ny0, nz0,
                 ax1, ay1, az1, nx1, ny1, nz1) = carry
                base = j * 16
                vx = buf[4, pl.ds(base, 16)]
                vy = buf[5, pl.ds(base, 16)]
                vz = buf[6, pl.ds(base, 16)]
                vi = buf[7, pl.ds(base, 16)]
                c0 = vi == float(i0)
                c1 = vi == float(i1)
                ax0 = jnp.maximum(ax0, jnp.where(c0, vx, _NEG))
                ay0 = jnp.maximum(ay0, jnp.where(c0, vy, _NEG))
                az0 = jnp.maximum(az0, jnp.where(c0, vz, _NEG))
                nx0 = jnp.minimum(nx0, jnp.where(c0, vx, _POS))
                ny0 = jnp.minimum(ny0, jnp.where(c0, vy, _POS))
                nz0 = jnp.minimum(nz0, jnp.where(c0, vz, _POS))
                ax1 = jnp.maximum(ax1, jnp.where(c1, vx, _NEG))
                ay1 = jnp.maximum(ay1, jnp.where(c1, vy, _NEG))
                az1 = jnp.maximum(az1, jnp.where(c1, vz, _NEG))
                nx1 = jnp.minimum(nx1, jnp.where(c1, vx, _POS))
                ny1 = jnp.minimum(ny1, jnp.where(c1, vy, _POS))
                nz1 = jnp.minimum(nz1, jnp.where(c1, vz, _POS))
                return (ax0, ay0, az0, nx0, ny0, nz0,
                        ax1, ay1, az1, nx1, ny1, nz1)

            neg = jnp.full((16,), _NEG, jnp.float32)
            pos = jnp.full((16,), _POS, jnp.float32)
            res = lax.fori_loop(
                0, _BN // 16, body,
                (neg, neg, neg, pos, pos, pos,
                 neg, neg, neg, pos, pos, pos))
            omax[i0, 0, :] = res[0]
            omax[i0, 1, :] = res[1]
            omax[i0, 2, :] = res[2]
            omin[i0, 0, :] = res[3]
            omin[i0, 1, :] = res[4]
            omin[i0, 2, :] = res[5]
            omax[i1, 0, :] = res[6]
            omax[i1, 1, :] = res[7]
            omax[i1, 2, :] = res[8]
            omin[i1, 0, :] = res[9]
            omin[i1, 1, :] = res[10]
            omin[i1, 2, :] = res[11]

    pltpu.sync_copy(omax, pmax_hbm.at[w])
    pltpu.sync_copy(omin, pmin_hbm.at[w])


def _segment_sc(aux):
    mesh = plsc.VectorSubcoreMesh(core_axis_name="c", subcore_axis_name="s")
    fn = functools.partial(
        pl.kernel, mesh=mesh,
        out_type=[
            jax.ShapeDtypeStruct((_NW, _I, 3, 16), jnp.float32),
            jax.ShapeDtypeStruct((_NW, _I, 3, 16), jnp.float32),
        ],
        scratch_types=[
            pltpu.VMEM((8, _BN), jnp.float32),
            pltpu.VMEM((_I, 3, 16), jnp.float32),
            pltpu.VMEM((_I, 3, 16), jnp.float32),
        ],
    )(_seg_sc_body)
    return fn(aux)


def _stage2_body(aux_ref, pmax_ref, pmin_ref, flow_ref, t_ref, yaw_ref,
                 vmaxs_ref, vmins_ref):
    ax = aux_ref[0]
    z = ax[3:4, :]
    px = ax[4:5, :]
    py = ax[5:6, :]
    pz = ax[6:7, :]
    ind = ax[7:8, :]

    @pl.when(pl.program_id(0) == 0)
    def _fold():
        rmax = pmax_ref[0]
        rmin = pmin_ref[0]
        for wx in range(1, _NW):
            rmax = jnp.maximum(rmax, pmax_ref[wx])
            rmin = jnp.minimum(rmin, pmin_ref[wx])
        vmaxs_ref[...] = jnp.max(rmax, axis=2)
        vmins_ref[...] = jnp.min(rmin, axis=2)

    zero = jnp.zeros_like(z)
    accx, accy, accz = zero, zero, zero

    def _center(c, i):
        mx = vmaxs_ref[i, c]
        mn = vmins_ref[i, c]
        return jnp.where(mx >= mn, (mx + mn) * 0.5, 0.0)

    for i in range(_I):
        cond = ind == float(i)
        accx = jnp.where(cond, _center(0, i), accx)
        accy = jnp.where(cond, _center(1, i), accy)
        accz = jnp.where(cond, _center(2, i), accz)

    dx = px - accx
    dy = py - accy
    dz = pz - accz

    a2 = z * z
    a = jnp.sqrt(a2 + 1e-12)
    small = a < 1e-4
    a_safe = jnp.where(small, 1.0, a)
    sin_term = jnp.where(small, 1.0 - a2 / 6.0, jnp.sin(a_safe) / a_safe)
    cos_term = jnp.where(small, 0.5 - a2 / 24.0,
                         (1.0 - jnp.cos(a_safe)) / (a_safe * a_safe))
    s = sin_term * z
    cm = 1.0 - cos_term * a2

    rx = cm * dx - s * dy
    ry = s * dx + cm * dy
    fx = (rx + accx + ax[0:1, :]) - px
    fy = (ry + accy + ax[1:2, :]) - py
    fz = (dz + accz + ax[2:3, :]) - pz
    flowT = jnp.concatenate([fx, fy, fz], axis=0)
    flow_ref[0] = flowT
    t_ref[0] = ax[0:3, :]
    yaw_ref[0] = ax[3:4, :]


def kernel(pc1, mask, W1, b1, W2, b2, W3, b3):
    pcT = jnp.transpose(pc1.reshape(_NB, _BN, 3), (0, 2, 1))
    mkT = jnp.transpose(mask.reshape(_NB, _BN, _I), (0, 2, 1))
    W1T, W2T, W3T = W1.T, W2.T, W3.T
    grid = (_NB,)

    mT, aux = pl.pallas_call(
        _stage1_body,
        grid=grid,
        in_specs=[
            pl.BlockSpec((1, 3, _BN), lambda i: (i, 0, 0)),
            pl.BlockSpec((1, _I, _BN), lambda i: (i, 0, 0)),
            pl.BlockSpec((_H, 3), lambda i: (0, 0)),
            pl.BlockSpec((_H,), lambda i: (0,)),
            pl.BlockSpec((_H, _H), lambda i: (0, 0)),
            pl.BlockSpec((_H,), lambda i: (0,)),
            pl.BlockSpec((4, _H), lambda i: (0, 0)),
            pl.BlockSpec((4,), lambda i: (0,)),
        ],
        out_specs=[
            pl.BlockSpec((1, _I, _BN), lambda i: (i, 0, 0)),
            pl.BlockSpec((1, 8, _BN), lambda i: (i, 0, 0)),
        ],
        out_shape=[
            jax.ShapeDtypeStruct((_NB, _I, _BN), jnp.float32),
            jax.ShapeDtypeStruct((_NB, 8, _BN), jnp.float32),
        ],
    )(pcT, mkT, W1T, b1, W2T, b2, W3T, b3)

    pmax, pmin = _segment_sc(aux)

    flow, t, yaw = pl.pallas_call(
        _stage2_body,
        grid=grid,
        in_specs=[
            pl.BlockSpec((1, 8, _BN), lambda i: (i, 0, 0)),
            pl.BlockSpec((_NW, _I, 3, 16), lambda i: (0, 0, 0, 0)),
            pl.BlockSpec((_NW, _I, 3, 16), lambda i: (0, 0, 0, 0)),
        ],
        out_specs=[
            pl.BlockSpec((1, 3, _BN), lambda i: (i, 0, 0)),
            pl.BlockSpec((1, 3, _BN), lambda i: (i, 0, 0)),
            pl.BlockSpec((1, 1, _BN), lambda i: (i, 0, 0)),
        ],
        out_shape=[
            jax.ShapeDtypeStruct((_NB, 3, _BN), jnp.float32),
            jax.ShapeDtypeStruct((_NB, 3, _BN), jnp.float32),
            jax.ShapeDtypeStruct((_NB, 1, _BN), jnp.float32),
        ],
        scratch_shapes=[
            pltpu.VMEM((_I, 3), jnp.float32),
            pltpu.VMEM((_I, 3), jnp.float32),
        ],
    )(aux, pmax, pmin)

    m = jnp.transpose(mT, (0, 2, 1)).reshape(1, _N, _I)
    flow = jnp.transpose(flow, (0, 2, 1)).reshape(1, _N, 3)
    t = jnp.transpose(t, (0, 2, 1)).reshape(1, _N, 3)
    yaw = jnp.transpose(yaw, (0, 2, 1)).reshape(1, _N, 1)
    return (flow, m, t, yaw)
